# baseline (speedup 1.0000x reference)
import functools

import jax
import jax.numpy as jnp
from jax import lax
from jax.experimental import pallas as pl
from jax.experimental.pallas import tpu as pltpu
from jax.experimental.pallas import tpu_sc as plsc

N = 10000
E = 320000
D = 128

NC = 2
NS = 16
NW = NC * NS
EW = E // NW
K = 128
NFULL = EW // K
REM = EW - NFULL * K
ROWS_MOST = 632
ROWS_LAST = N - (NS - 1) * ROWS_MOST


def _sc_segment_sum(x, edge_index, zeros_acc, zeros_deg, ones_rows):
  mesh = plsc.VectorSubcoreMesh(core_axis_name="c", subcore_axis_name="s")

  @functools.partial(
      pl.kernel,
      out_type=[
          jax.ShapeDtypeStruct((NC, N, D), jnp.float32),
          jax.ShapeDtypeStruct((NC, N, 16), jnp.float32),
      ],
      mesh=mesh,
      scratch_types=[
          [pltpu.VMEM((K,), jnp.int32)] * 4,
          [pltpu.VMEM((K,), jnp.int32)] * 4,
          [pltpu.VMEM((K, D), jnp.float32)] * 2,
          pltpu.VMEM((K, 16), jnp.float32),
          pltpu.VMEM((REM,), jnp.int32),
          pltpu.VMEM((REM,), jnp.int32),
          pltpu.VMEM((REM, D), jnp.float32),
          pltpu.VMEM((REM, 16), jnp.float32),
          pltpu.VMEM_SHARED((N, D), jnp.float32),
          pltpu.VMEM_SHARED((N, 16), jnp.float32),
          [pltpu.SemaphoreType.DMA] * 2,
          [pltpu.SemaphoreType.DMA] * 2,
          [pltpu.SemaphoreType.DMA] * 4,
          pltpu.SemaphoreType.DMA,
      ],
      compiler_params=pltpu.CompilerParams(use_tc_tiling_on_sc=False),
  )
  def k(x_hbm, edge_hbm, zacc_hbm, zdeg_hbm, ones_hbm, acc_out,
        deg_out, srcs, tgts, rowss, ones_v, src_r, tgt_r, rows_r, ones_r,
        acc_s, deg_s, gsems, ssems, isems, rsem):
    src_hbm = edge_hbm.at[1]
    tgt_hbm = edge_hbm.at[0]
    c = lax.axis_index("c")
    s = lax.axis_index("s")
    wid = c * NS + s

    pltpu.sync_copy(ones_hbm, ones_v)
    pltpu.sync_copy(ones_hbm.at[pl.ds(0, REM)], ones_r)

    r0 = s * ROWS_MOST

    def for_my_rows(f):
      @pl.when(s < NS - 1)
      def _():
        f(r0, ROWS_MOST)
      @pl.when(s == NS - 1)
      def _():
        f(r0, ROWS_LAST)

    def zero_rows(r, n):
      pltpu.sync_copy(zacc_hbm.at[pl.ds(0, n)], acc_s.at[pl.ds(r, n)])
      pltpu.sync_copy(zdeg_hbm.at[pl.ds(0, n)], deg_s.at[pl.ds(r, n)])
    for_my_rows(zero_rows)
    plsc.subcore_barrier()

    base = wid * EW

    def idx_start(off, i):
      pltpu.async_copy(src_hbm.at[pl.ds(off, K)], srcs[i], isems[i])
      pltpu.async_copy(tgt_hbm.at[pl.ds(off, K)], tgts[i], isems[i])

    def idx_wait(i):
      pltpu.make_async_copy(src_hbm.at[pl.ds(0, K)], srcs[i],
                            isems[i]).wait()
      pltpu.make_async_copy(tgt_hbm.at[pl.ds(0, K)], tgts[i],
                            isems[i]).wait()

    def gather_start(r, i):
      pltpu.async_copy(x_hbm.at[srcs[i]], rowss[r], gsems[r])

    def gather_wait(r, i):
      pltpu.make_async_copy(x_hbm.at[srcs[i]], rowss[r], gsems[r]).wait()

    def scatter_start(r, i):
      pltpu.async_copy(rowss[r], acc_s.at[tgts[i]], ssems[r], add=True)
      pltpu.async_copy(ones_v, deg_s.at[tgts[i]], ssems[r], add=True)

    def scatter_wait(r, i):
      pltpu.make_async_copy(rowss[r], acc_s.at[tgts[i]], ssems[r]).wait()
      pltpu.make_async_copy(ones_v, deg_s.at[tgts[i]], ssems[r]).wait()

    idx_start(base, 0)
    idx_start(base + K, 1)
    idx_wait(0)
    gather_start(0, 0)

    def emit_step(j_expr, u, has_next, has_next2):
      r, i = u % 2, u % 4
      gather_wait(r, i)
      if has_next:
        idx_wait((u + 1) % 4)
        gather_start((u + 1) % 2, (u + 1) % 4)
      scatter_start(r, i)
      scatter_wait(r, i)
      if has_next2:
        idx_start(j_expr + 2 * K, (u + 2) % 4)

    for u in range(4):
      emit_step(base + u * K, u, u + 1 < NFULL, u + 2 < NFULL)

    def group(t, _):
      j0 = base + 4 * t * K
      for u in range(4):
        emit_step(j0 + u * K, u, True, True)
      return 0
    lax.fori_loop(1, NFULL // 4, group, 0)

    for u in range(4 * (NFULL // 4), NFULL):
      emit_step(base + u * K, u % 4, u + 1 < NFULL, u + 2 < NFULL)

    off_r = base + NFULL * K
    pltpu.sync_copy(src_hbm.at[pl.ds(off_r, REM)], src_r)
    pltpu.sync_copy(tgt_hbm.at[pl.ds(off_r, REM)], tgt_r)
    pltpu.async_copy(x_hbm.at[src_r], rows_r, rsem).wait()
    pltpu.sync_copy(rows_r, acc_s.at[tgt_r], add=True)
    pltpu.sync_copy(ones_r, deg_s.at[tgt_r], add=True)

    plsc.subcore_barrier()

    def publish_rows(r, n):
      pltpu.sync_copy(acc_s.at[pl.ds(r, n)], acc_out.at[c, pl.ds(r, n)])
      pltpu.sync_copy(deg_s.at[pl.ds(r, n)], deg_out.at[c, pl.ds(r, n)])
    for_my_rows(publish_rows)

  return k(x, edge_index, zeros_acc, zeros_deg, ones_rows)


BLK = 2000


def _tc_pre_kernel(x_ref, wm_ref, bm_ref, w1_ref, b1_ref, qa_ref, r_ref,
                   a_ref):
  f32 = jnp.float32
  xv = x_ref[...]
  wm = wm_ref[...]
  w1a = w1_ref[0:3 * D]
  q = jnp.dot(xv, wm[D:], preferred_element_type=f32) + bm_ref[...]
  qa_ref[...] = jnp.dot(q, w1a, preferred_element_type=f32)
  r_ref[...] = jnp.dot(xv, w1_ref[3 * D:], preferred_element_type=f32) \
      + b1_ref[...]
  @pl.when(pl.program_id(0) == 0)
  def _():
    a_ref[...] = jnp.dot(wm[:D], w1a, preferred_element_type=f32)


def _tc_pre(x, W_msg, b_msg, W_u1, b_u1):
  grid = (N // BLK,)
  full = lambda *shape: pl.BlockSpec(shape, lambda i: (0,) * len(shape))
  return pl.pallas_call(
      _tc_pre_kernel,
      grid=grid,
      in_specs=[
          pl.BlockSpec((BLK, D), lambda i: (i, 0)),
          full(2 * D, 3 * D),
          full(1, 3 * D),
          full(4 * D, 2 * D),
          full(1, 2 * D),
      ],
      out_specs=[
          pl.BlockSpec((BLK, 2 * D), lambda i: (i, 0)),
          pl.BlockSpec((BLK, 2 * D), lambda i: (i, 0)),
          full(D, 2 * D),
      ],
      out_shape=[
          jax.ShapeDtypeStruct((N, 2 * D), jnp.float32),
          jax.ShapeDtypeStruct((N, 2 * D), jnp.float32),
          jax.ShapeDtypeStruct((D, 2 * D), jnp.float32),
      ],
  )(x, W_msg, b_msg, W_u1, b_u1)


def _tc_post_kernel(acc_ref, deg_ref, x_ref, qa_ref, r_ref, a_ref, w2_ref,
                    b2_ref, out_ref):
  f32 = jnp.float32
  S = acc_ref[0] + acc_ref[1]
  deg = deg_ref[0, :, 0:1] + deg_ref[1, :, 0:1]
  h = jnp.dot(S, a_ref[...], preferred_element_type=f32) \
      + deg * qa_ref[...] + r_ref[...]
  h = jnp.maximum(h, 0.0)
  out_ref[...] = x_ref[...] \
      + jnp.dot(h, w2_ref[...], preferred_element_type=f32) + b2_ref[...]


def _tc_post(acc, deg, x, qa, r, a, W_u2, b_u2):
  grid = (N // BLK,)
  full = lambda *shape: pl.BlockSpec(shape, lambda i: (0,) * len(shape))
  return pl.pallas_call(
      _tc_post_kernel,
      grid=grid,
      in_specs=[
          pl.BlockSpec((NC, BLK, D), lambda i: (0, i, 0)),
          pl.BlockSpec((NC, BLK, 16), lambda i: (0, i, 0)),
          pl.BlockSpec((BLK, D), lambda i: (i, 0)),
          pl.BlockSpec((BLK, 2 * D), lambda i: (i, 0)),
          pl.BlockSpec((BLK, 2 * D), lambda i: (i, 0)),
          full(D, 2 * D),
          full(2 * D, D),
          full(1, D),
      ],
      out_specs=pl.BlockSpec((BLK, D), lambda i: (i, 0)),
      out_shape=jax.ShapeDtypeStruct((N, D), jnp.float32),
  )(acc, deg, x, qa, r, a, W_u2, b_u2)


def kernel(x, edge_index, W_msg, b_msg, W_u1, b_u1, W_u2, b_u2):
  zeros_acc = jnp.zeros((ROWS_MOST, D), jnp.float32)
  zeros_deg = jnp.zeros((ROWS_MOST, 16), jnp.float32)
  ones_rows = jnp.ones((K, 16), jnp.float32)
  acc, deg = _sc_segment_sum(x, edge_index, zeros_acc, zeros_deg, ones_rows)
  qa, r, a = _tc_pre(x, W_msg, b_msg.reshape(1, -1), W_u1,
                     b_u1.reshape(1, -1))
  return _tc_post(acc, deg, x, qa, r, a, W_u2, b_u2.reshape(1, -1))


# speedup vs baseline: 1.0088x; 1.0088x over previous
import functools

import jax
import jax.numpy as jnp
from jax import lax
from jax.experimental import pallas as pl
from jax.experimental.pallas import tpu as pltpu
from jax.experimental.pallas import tpu_sc as plsc

N = 10000
E = 320000
D = 128
NP = 10240

NC = 2
NS = 16
NW = NC * NS
EW = E // NW
K = 128
NFULL = EW // K
REM = EW - NFULL * K
RPT = NP // NS


def _sc_segment_sum(x, edge_index, zeros_acc):
  mesh = plsc.VectorSubcoreMesh(core_axis_name="c", subcore_axis_name="s")

  @functools.partial(
      pl.kernel,
      out_type=[
          jax.ShapeDtypeStruct((NC, NP, D), jnp.float32),
          jax.ShapeDtypeStruct((NW, NP), jnp.float32),
      ],
      mesh=mesh,
      scratch_types=[
          [pltpu.VMEM((K,), jnp.int32)] * 4,
          [pltpu.VMEM((K,), jnp.int32)] * 4,
          [pltpu.VMEM((K, D), jnp.float32)] * 2,
          pltpu.VMEM((NP,), jnp.float32),
          pltpu.VMEM((REM,), jnp.int32),
          pltpu.VMEM((REM,), jnp.int32),
          pltpu.VMEM((REM, D), jnp.float32),
          pltpu.VMEM_SHARED((NP, D), jnp.float32),
          [pltpu.SemaphoreType.DMA] * 2,
          [pltpu.SemaphoreType.DMA] * 2,
          [pltpu.SemaphoreType.DMA] * 4,
          pltpu.SemaphoreType.DMA,
      ],
      compiler_params=pltpu.CompilerParams(use_tc_tiling_on_sc=False,
                                           needs_layout_passes=False),
  )
  def k(x_hbm, edge_hbm, zacc_hbm, acc_out, deg_out,
        srcs, tgts, rowss, hist_v, src_r, tgt_r, rows_r,
        acc_s, gsems, ssems, isems, rsem):
    src_hbm = edge_hbm.at[1]
    tgt_hbm = edge_hbm.at[0]
    c = lax.axis_index("c")
    s = lax.axis_index("s")
    wid = c * NS + s

    zeros16 = jnp.zeros((16,), jnp.float32)

    def zero_hist(i, _):
      hist_v[pl.ds(i * 16, 16)] = zeros16
      return 0
    lax.fori_loop(0, NP // 16, zero_hist, 0)

    ones16 = jnp.ones((16,), jnp.float32)

    def hist_update(tgt_b, groups):
      for g in range(groups):
        idx = tgt_b[pl.ds(g * 16, 16)]
        plsc.addupdate_scatter(hist_v, [idx], ones16)

    r0 = s * RPT
    pltpu.sync_copy(zacc_hbm, acc_s.at[pl.ds(r0, RPT)])
    plsc.subcore_barrier()

    base = wid * EW

    def idx_start(off, i):
      pltpu.async_copy(src_hbm.at[pl.ds(off, K)], srcs[i], isems[i])
      pltpu.async_copy(tgt_hbm.at[pl.ds(off, K)], tgts[i], isems[i])

    def idx_wait(i):
      pltpu.make_async_copy(src_hbm.at[pl.ds(0, K)], srcs[i],
                            isems[i]).wait()
      pltpu.make_async_copy(tgt_hbm.at[pl.ds(0, K)], tgts[i],
                            isems[i]).wait()

    def gather_start(r, i):
      pltpu.async_copy(x_hbm.at[srcs[i]], rowss[r], gsems[r])

    def gather_wait(r, i):
      pltpu.make_async_copy(x_hbm.at[srcs[i]], rowss[r], gsems[r]).wait()

    def scatter_start(r, i):
      pltpu.async_copy(rowss[r], acc_s.at[tgts[i]], ssems[r], add=True)

    def scatter_wait(r, i):
      pltpu.make_async_copy(rowss[r], acc_s.at[tgts[i]], ssems[r]).wait()

    idx_start(base, 0)
    idx_start(base + K, 1)
    idx_wait(0)
    gather_start(0, 0)

    def emit_step(j_expr, u, has_next, has_next2):
      r, i = u % 2, u % 4
      gather_wait(r, i)
      if has_next:
        idx_wait((u + 1) % 4)
        gather_start((u + 1) % 2, (u + 1) % 4)
      scatter_start(r, i)
      hist_update(tgts[i], K // 16)
      scatter_wait(r, i)
      if has_next2:
        idx_start(j_expr + 2 * K, (u + 2) % 4)

    for u in range(4):
      emit_step(base + u * K, u, u + 1 < NFULL, u + 2 < NFULL)

    def group(t, _):
      j0 = base + 4 * t * K
      for u in range(4):
        emit_step(j0 + u * K, u, True, True)
      return 0
    lax.fori_loop(1, NFULL // 4, group, 0)

    for u in range(4 * (NFULL // 4), NFULL):
      emit_step(base + u * K, u % 4, u + 1 < NFULL, u + 2 < NFULL)

    off_r = base + NFULL * K
    pltpu.sync_copy(src_hbm.at[pl.ds(off_r, REM)], src_r)
    pltpu.sync_copy(tgt_hbm.at[pl.ds(off_r, REM)], tgt_r)
    pltpu.async_copy(x_hbm.at[src_r], rows_r, rsem).wait()
    pltpu.sync_copy(rows_r, acc_s.at[tgt_r], add=True)
    hist_update(tgt_r, REM // 16)

    pltpu.sync_copy(hist_v, deg_out.at[wid])
    plsc.subcore_barrier()
    pltpu.sync_copy(acc_s.at[pl.ds(r0, RPT)], acc_out.at[c, pl.ds(r0, RPT)])

  return k(x, edge_index, zeros_acc)


BLK = 2048


H = 2 * D


def _dotg(lhs, rhs, l_dim, r_dim):
  return lax.dot_general(lhs, rhs, (((l_dim,), (r_dim,)), ((), ())),
                         preferred_element_type=jnp.float32)


def _tc_pre_kernel(x_ref, wm_ref, bm_ref, w1_ref, b1_ref, qa_ref, r_ref,
                   a_ref):
  f32 = jnp.float32
  xv = x_ref[...]
  wm = wm_ref[...]
  w1a = w1_ref[0:3 * D]
  q = jnp.dot(xv, wm[D:], preferred_element_type=f32) + bm_ref[...]
  qa_ref[...] = jnp.dot(q, w1a, preferred_element_type=f32)
  r_ref[...] = jnp.dot(xv, w1_ref[3 * D:], preferred_element_type=f32) \
      + b1_ref[...]
  @pl.when(pl.program_id(0) == 0)
  def _():
    a_ref[...] = jnp.dot(wm[:D], w1a, preferred_element_type=f32)


def _tc_pre(x, W_msg, b_msg, W_u1, b_u1):
  grid = (NP // BLK,)
  full = lambda *shape: pl.BlockSpec(shape, lambda i: (0,) * len(shape))
  return pl.pallas_call(
      _tc_pre_kernel,
      grid=grid,
      in_specs=[
          pl.BlockSpec((BLK, D), lambda i: (i, 0)),
          full(2 * D, 3 * D),
          full(1, 3 * D),
          full(4 * D, H),
          full(1, H),
      ],
      out_specs=[
          pl.BlockSpec((BLK, H), lambda i: (i, 0)),
          pl.BlockSpec((BLK, H), lambda i: (i, 0)),
          full(D, H),
      ],
      out_shape=[
          jax.ShapeDtypeStruct((NP, H), jnp.float32),
          jax.ShapeDtypeStruct((NP, H), jnp.float32),
          jax.ShapeDtypeStruct((D, H), jnp.float32),
      ],
  )(x, W_msg, b_msg, W_u1, b_u1)


def _tc_post_kernel(acc_ref, deg_ref, x_ref, qa_ref, r_ref, a_ref, w2_ref,
                    b2_ref, out_ref):
  f32 = jnp.float32
  S = acc_ref[0] + acc_ref[1]
  deg = _dotg(deg_ref[...], jnp.ones((NW, 1), f32), 0, 0)
  h = jnp.dot(S, a_ref[...], preferred_element_type=f32) \
      + deg * qa_ref[...] + r_ref[...]
  h = jnp.maximum(h, 0.0)
  out_ref[...] = x_ref[...] \
      + jnp.dot(h, w2_ref[...], preferred_element_type=f32) + b2_ref[...]


def _tc_post(acc, deg, x, qa, r, a, W_u2, b_u2):
  grid = (NP // BLK,)
  full = lambda *shape: pl.BlockSpec(shape, lambda i: (0,) * len(shape))
  return pl.pallas_call(
      _tc_post_kernel,
      grid=grid,
      in_specs=[
          pl.BlockSpec((NC, BLK, D), lambda i: (0, i, 0)),
          pl.BlockSpec((NW, BLK), lambda i: (0, i)),
          pl.BlockSpec((BLK, D), lambda i: (i, 0)),
          pl.BlockSpec((BLK, H), lambda i: (i, 0)),
          pl.BlockSpec((BLK, H), lambda i: (i, 0)),
          full(D, H),
          full(H, D),
          full(1, D),
      ],
      out_specs=pl.BlockSpec((BLK, D), lambda i: (i, 0)),
      out_shape=jax.ShapeDtypeStruct((NP, D), jnp.float32),
  )(acc, deg, x, qa, r, a, W_u2, b_u2)


def kernel(x, edge_index, W_msg, b_msg, W_u1, b_u1, W_u2, b_u2):
  xp = jnp.concatenate([x, jnp.zeros((NP - N, D), jnp.float32)], axis=0)
  zeros_acc = jnp.zeros((RPT, D), jnp.float32)
  acc, deg = _sc_segment_sum(xp, edge_index, zeros_acc)
  qa, r, a = _tc_pre(xp, W_msg, b_msg.reshape(1, -1), W_u1,
                     b_u1.reshape(1, -1))
  out = _tc_post(acc, deg, xp, qa, r, a, W_u2, b_u2.reshape(1, -1))
  return out[:N]


# speedup vs baseline: 1.0590x; 1.0497x over previous
import functools

import jax
import jax.numpy as jnp
from jax import lax
from jax.experimental import pallas as pl
from jax.experimental.pallas import tpu as pltpu
from jax.experimental.pallas import tpu_sc as plsc

N = 10000
E = 320000
D = 128
NP = 10240

NC = 2
NS = 16
NW = NC * NS
EW = E // NW
K = 128
NFULL = EW // K
REM = EW - NFULL * K
RPT = NP // NS


def _sc_segment_sum(x, edge_index, zeros_acc):
  mesh = plsc.VectorSubcoreMesh(core_axis_name="c", subcore_axis_name="s")

  @functools.partial(
      pl.kernel,
      out_type=[
          jax.ShapeDtypeStruct((NC, NP, D), jnp.float32),
          jax.ShapeDtypeStruct((NW, NP), jnp.float32),
      ],
      mesh=mesh,
      scratch_types=[
          [pltpu.VMEM((K,), jnp.int32)] * 4,
          [pltpu.VMEM((K,), jnp.int32)] * 4,
          [pltpu.VMEM((K, D), jnp.float32)] * 2,
          pltpu.VMEM((NP,), jnp.float32),
          pltpu.VMEM((REM,), jnp.int32),
          pltpu.VMEM((REM,), jnp.int32),
          pltpu.VMEM((REM, D), jnp.float32),
          pltpu.VMEM_SHARED((NP, D), jnp.float32),
          [pltpu.SemaphoreType.DMA] * 2,
          [pltpu.SemaphoreType.DMA] * 2,
          [pltpu.SemaphoreType.DMA] * 4,
          pltpu.SemaphoreType.DMA,
      ],
      compiler_params=pltpu.CompilerParams(use_tc_tiling_on_sc=False,
                                           needs_layout_passes=False),
  )
  def k(x_hbm, edge_hbm, zacc_hbm, acc_out, deg_out,
        srcs, tgts, rowss, hist_v, src_r, tgt_r, rows_r,
        acc_s, gsems, ssems, isems, rsem):
    src_hbm = edge_hbm.at[1]
    tgt_hbm = edge_hbm.at[0]
    c = lax.axis_index("c")
    s = lax.axis_index("s")
    wid = c * NS + s

    zeros16 = jnp.zeros((16,), jnp.float32)

    def zero_hist(i, _):
      hist_v[pl.ds(i * 16, 16)] = zeros16
      return 0
    lax.fori_loop(0, NP // 16, zero_hist, 0)

    ones16 = jnp.ones((16,), jnp.float32)

    def hist_update(tgt_b, groups):
      for g in range(groups):
        idx = tgt_b[pl.ds(g * 16, 16)]
        plsc.addupdate_scatter(hist_v, [idx], ones16)

    r0 = s * RPT
    pltpu.sync_copy(zacc_hbm, acc_s.at[pl.ds(r0, RPT)])
    plsc.subcore_barrier()

    base = wid * EW

    def idx_start(off, i):
      pltpu.async_copy(src_hbm.at[pl.ds(off, K)], srcs[i], isems[i])
      pltpu.async_copy(tgt_hbm.at[pl.ds(off, K)], tgts[i], isems[i])

    def idx_wait(i):
      pltpu.make_async_copy(src_hbm.at[pl.ds(0, K)], srcs[i],
                            isems[i]).wait()
      pltpu.make_async_copy(tgt_hbm.at[pl.ds(0, K)], tgts[i],
                            isems[i]).wait()

    def gather_start(r, i):
      pltpu.async_copy(x_hbm.at[srcs[i]], rowss[r], gsems[r])

    def gather_wait(r, i):
      pltpu.make_async_copy(x_hbm.at[srcs[i]], rowss[r], gsems[r]).wait()

    def scatter_start(r, i):
      pltpu.async_copy(rowss[r], acc_s.at[tgts[i]], ssems[r], add=True)

    def scatter_wait(r, i):
      pltpu.make_async_copy(rowss[r], acc_s.at[tgts[i]], ssems[r]).wait()

    idx_start(base, 0)
    idx_start(base + K, 1)
    idx_wait(0)
    gather_start(0, 0)

    def emit_step(j_expr, u, has_next, has_next2):
      r, i = u % 2, u % 4
      if has_next2:
        idx_start(j_expr + 2 * K, (u + 2) % 4)
      gather_wait(r, i)
      if has_next:
        idx_wait((u + 1) % 4)
        gather_start((u + 1) % 2, (u + 1) % 4)
      scatter_start(r, i)
      hist_update(tgts[i], K // 16)
      scatter_wait(r, i)

    for u in range(4):
      emit_step(base + u * K, u, u + 1 < NFULL, u + 2 < NFULL)

    def group(t, _):
      j0 = base + 4 * t * K
      for u in range(4):
        emit_step(j0 + u * K, u, True, True)
      return 0
    lax.fori_loop(1, NFULL // 4, group, 0)

    for u in range(4 * (NFULL // 4), NFULL):
      emit_step(base + u * K, u % 4, u + 1 < NFULL, u + 2 < NFULL)

    off_r = base + NFULL * K
    pltpu.sync_copy(src_hbm.at[pl.ds(off_r, REM)], src_r)
    pltpu.sync_copy(tgt_hbm.at[pl.ds(off_r, REM)], tgt_r)
    pltpu.async_copy(x_hbm.at[src_r], rows_r, rsem).wait()
    pltpu.sync_copy(rows_r, acc_s.at[tgt_r], add=True)
    hist_update(tgt_r, REM // 16)

    pltpu.sync_copy(hist_v, deg_out.at[wid])
    plsc.subcore_barrier()
    pltpu.sync_copy(acc_s.at[pl.ds(r0, RPT)], acc_out.at[c, pl.ds(r0, RPT)])

  return k(x, edge_index, zeros_acc)


BLK = 2048


H = 2 * D


def _dotg(lhs, rhs, l_dim, r_dim):
  return lax.dot_general(lhs, rhs, (((l_dim,), (r_dim,)), ((), ())),
                         preferred_element_type=jnp.float32)


def _tc_pre_kernel(x_ref, wm_ref, bm_ref, w1_ref, b1_ref, qa_ref, r_ref,
                   a_ref):
  f32 = jnp.float32
  xv = x_ref[...]
  wm = wm_ref[...]
  w1a = w1_ref[0:3 * D]
  q = jnp.dot(xv, wm[D:], preferred_element_type=f32) + bm_ref[...]
  qa_ref[...] = jnp.dot(q, w1a, preferred_element_type=f32)
  r_ref[...] = jnp.dot(xv, w1_ref[3 * D:], preferred_element_type=f32) \
      + b1_ref[...]
  @pl.when(pl.program_id(0) == 0)
  def _():
    a_ref[...] = jnp.dot(wm[:D], w1a, preferred_element_type=f32)


def _tc_pre(x, W_msg, b_msg, W_u1, b_u1):
  grid = (NP // BLK,)
  full = lambda *shape: pl.BlockSpec(shape, lambda i: (0,) * len(shape))
  return pl.pallas_call(
      _tc_pre_kernel,
      grid=grid,
      in_specs=[
          pl.BlockSpec((BLK, D), lambda i: (i, 0)),
          full(2 * D, 3 * D),
          full(1, 3 * D),
          full(4 * D, H),
          full(1, H),
      ],
      out_specs=[
          pl.BlockSpec((BLK, H), lambda i: (i, 0)),
          pl.BlockSpec((BLK, H), lambda i: (i, 0)),
          full(D, H),
      ],
      out_shape=[
          jax.ShapeDtypeStruct((NP, H), jnp.float32),
          jax.ShapeDtypeStruct((NP, H), jnp.float32),
          jax.ShapeDtypeStruct((D, H), jnp.float32),
      ],
  )(x, W_msg, b_msg, W_u1, b_u1)


def _tc_post_kernel(acc_ref, deg_ref, x_ref, qa_ref, r_ref, a_ref, w2_ref,
                    b2_ref, out_ref):
  f32 = jnp.float32
  S = acc_ref[0] + acc_ref[1]
  deg = _dotg(deg_ref[...], jnp.ones((NW, 1), f32), 0, 0)
  h = jnp.dot(S, a_ref[...], preferred_element_type=f32) \
      + deg * qa_ref[...] + r_ref[...]
  h = jnp.maximum(h, 0.0)
  out_ref[...] = x_ref[...] \
      + jnp.dot(h, w2_ref[...], preferred_element_type=f32) + b2_ref[...]


def _tc_post(acc, deg, x, qa, r, a, W_u2, b_u2):
  grid = (NP // BLK,)
  full = lambda *shape: pl.BlockSpec(shape, lambda i: (0,) * len(shape))
  return pl.pallas_call(
      _tc_post_kernel,
      grid=grid,
      in_specs=[
          pl.BlockSpec((NC, BLK, D), lambda i: (0, i, 0)),
          pl.BlockSpec((NW, BLK), lambda i: (0, i)),
          pl.BlockSpec((BLK, D), lambda i: (i, 0)),
          pl.BlockSpec((BLK, H), lambda i: (i, 0)),
          pl.BlockSpec((BLK, H), lambda i: (i, 0)),
          full(D, H),
          full(H, D),
          full(1, D),
      ],
      out_specs=pl.BlockSpec((BLK, D), lambda i: (i, 0)),
      out_shape=jax.ShapeDtypeStruct((N, D), jnp.float32),
  )(acc, deg, x, qa, r, a, W_u2, b_u2)


def kernel(x, edge_index, W_msg, b_msg, W_u1, b_u1, W_u2, b_u2):
  zeros_acc = jnp.zeros((RPT, D), jnp.float32)
  acc, deg = _sc_segment_sum(x, edge_index, zeros_acc)
  qa, r, a = _tc_pre(x, W_msg, b_msg.reshape(1, -1), W_u1,
                     b_u1.reshape(1, -1))
  return _tc_post(acc, deg, x, qa, r, a, W_u2, b_u2.reshape(1, -1))


# speedup vs baseline: 1.0852x; 1.0248x over previous
import functools

import jax
import jax.numpy as jnp
from jax import lax
from jax.experimental import pallas as pl
from jax.experimental.pallas import tpu as pltpu
from jax.experimental.pallas import tpu_sc as plsc

N = 10000
E = 320000
D = 128
NP = 10240

NC = 2
NS = 16
NW = NC * NS
EW = E // NW
K = 128
NFULL = EW // K
REM = EW - NFULL * K
RPT = NP // NS


def _sc_segment_sum(x, edge_index, zeros_acc):
  mesh = plsc.VectorSubcoreMesh(core_axis_name="c", subcore_axis_name="s")

  @functools.partial(
      pl.kernel,
      out_type=[
          jax.ShapeDtypeStruct((NC, NP, D), jnp.float32),
          jax.ShapeDtypeStruct((NW, NP), jnp.float32),
      ],
      mesh=mesh,
      scratch_types=[
          [pltpu.VMEM((K,), jnp.int32)] * 4,
          [pltpu.VMEM((K,), jnp.int32)] * 4,
          [pltpu.VMEM((K, D), jnp.float32)] * 2,
          pltpu.VMEM((NP,), jnp.float32),
          pltpu.VMEM((REM,), jnp.int32),
          pltpu.VMEM((REM,), jnp.int32),
          pltpu.VMEM((REM, D), jnp.float32),
          pltpu.VMEM_SHARED((NP, D), jnp.float32),
          [pltpu.SemaphoreType.DMA] * 2,
          [pltpu.SemaphoreType.DMA] * 2,
          [pltpu.SemaphoreType.DMA] * 4,
          pltpu.SemaphoreType.DMA,
      ],
      compiler_params=pltpu.CompilerParams(use_tc_tiling_on_sc=False,
                                           needs_layout_passes=False),
  )
  def k(x_hbm, edge_hbm, zacc_hbm, acc_out, deg_out,
        srcs, tgts, rowss, hist_v, src_r, tgt_r, rows_r,
        acc_s, gsems, ssems, isems, rsem):
    src_hbm = edge_hbm.at[1]
    tgt_hbm = edge_hbm.at[0]
    c = lax.axis_index("c")
    s = lax.axis_index("s")
    wid = c * NS + s

    zeros16 = jnp.zeros((16,), jnp.float32)
    ones16 = jnp.ones((16,), jnp.float32)

    def hist_update(tgt_b, groups):
      for g in range(groups):
        idx = tgt_b[pl.ds(g * 16, 16)]
        plsc.addupdate_scatter(hist_v, [idx], ones16)

    base = wid * EW
    r0 = s * RPT

    def idx_start(off, i):
      pltpu.async_copy(src_hbm.at[pl.ds(off, K)], srcs[i], isems[i])
      pltpu.async_copy(tgt_hbm.at[pl.ds(off, K)], tgts[i], isems[i])

    def idx_wait(i):
      pltpu.make_async_copy(src_hbm.at[pl.ds(0, K)], srcs[i],
                            isems[i]).wait()
      pltpu.make_async_copy(tgt_hbm.at[pl.ds(0, K)], tgts[i],
                            isems[i]).wait()

    def gather_start(r, i):
      pltpu.async_copy(x_hbm.at[srcs[i]], rowss[r], gsems[r])

    def gather_wait(r, i):
      pltpu.make_async_copy(x_hbm.at[srcs[i]], rowss[r], gsems[r]).wait()

    def scatter_start(r, i):
      pltpu.async_copy(rowss[r], acc_s.at[tgts[i]], ssems[r], add=True)

    def scatter_wait(r, i):
      pltpu.make_async_copy(rowss[r], acc_s.at[tgts[i]], ssems[r]).wait()

    idx_start(base, 0)
    idx_start(base + K, 1)
    pltpu.async_copy(zacc_hbm, acc_s.at[pl.ds(r0, RPT)], rsem)
    idx_wait(0)
    gather_start(0, 0)

    def zero_hist(i, _):
      hist_v[pl.ds(i * 16, 16)] = zeros16
      return 0
    lax.fori_loop(0, NP // 16, zero_hist, 0)

    pltpu.make_async_copy(zacc_hbm, acc_s.at[pl.ds(r0, RPT)], rsem).wait()
    plsc.subcore_barrier()

    def emit_step(j_expr, u, has_next, has_next2):
      r, i = u % 2, u % 4
      if has_next2:
        idx_start(j_expr + 2 * K, (u + 2) % 4)
      gather_wait(r, i)
      if has_next:
        idx_wait((u + 1) % 4)
        gather_start((u + 1) % 2, (u + 1) % 4)
      scatter_start(r, i)
      hist_update(tgts[i], K // 16)
      scatter_wait(r, i)

    for u in range(4):
      emit_step(base + u * K, u, u + 1 < NFULL, u + 2 < NFULL)

    def group(t, _):
      j0 = base + 4 * t * K
      for u in range(4):
        emit_step(j0 + u * K, u, True, True)
      return 0
    lax.fori_loop(1, NFULL // 4, group, 0)

    for u in range(4 * (NFULL // 4), NFULL):
      emit_step(base + u * K, u % 4, u + 1 < NFULL, u + 2 < NFULL)

    off_r = base + NFULL * K
    pltpu.sync_copy(src_hbm.at[pl.ds(off_r, REM)], src_r)
    pltpu.sync_copy(tgt_hbm.at[pl.ds(off_r, REM)], tgt_r)
    pltpu.async_copy(x_hbm.at[src_r], rows_r, rsem).wait()
    pltpu.sync_copy(rows_r, acc_s.at[tgt_r], add=True)
    hist_update(tgt_r, REM // 16)

    pltpu.sync_copy(hist_v, deg_out.at[wid])
    plsc.subcore_barrier()
    pltpu.sync_copy(acc_s.at[pl.ds(r0, RPT)], acc_out.at[c, pl.ds(r0, RPT)])

  return k(x, edge_index, zeros_acc)


BLK = 2048


H = 2 * D


def _dotg(lhs, rhs, l_dim, r_dim):
  return lax.dot_general(lhs, rhs, (((l_dim,), (r_dim,)), ((), ())),
                         preferred_element_type=jnp.float32)


def _tc_pre_kernel(x_ref, wm_ref, bm_ref, w1_ref, b1_ref, qa_ref, r_ref,
                   a_ref):
  f32 = jnp.float32
  xv = x_ref[...]
  wm = wm_ref[...]
  w1a = w1_ref[0:3 * D]
  q = jnp.dot(xv, wm[D:], preferred_element_type=f32) + bm_ref[...]
  qa_ref[...] = jnp.dot(q, w1a, preferred_element_type=f32)
  r_ref[...] = jnp.dot(xv, w1_ref[3 * D:], preferred_element_type=f32) \
      + b1_ref[...]
  @pl.when(pl.program_id(0) == 0)
  def _():
    a_ref[...] = jnp.dot(wm[:D], w1a, preferred_element_type=f32)


def _tc_pre(x, W_msg, b_msg, W_u1, b_u1):
  grid = (NP // BLK,)
  full = lambda *shape: pl.BlockSpec(shape, lambda i: (0,) * len(shape))
  return pl.pallas_call(
      _tc_pre_kernel,
      grid=grid,
      in_specs=[
          pl.BlockSpec((BLK, D), lambda i: (i, 0)),
          full(2 * D, 3 * D),
          full(1, 3 * D),
          full(4 * D, H),
          full(1, H),
      ],
      out_specs=[
          pl.BlockSpec((BLK, H), lambda i: (i, 0)),
          pl.BlockSpec((BLK, H), lambda i: (i, 0)),
          full(D, H),
      ],
      out_shape=[
          jax.ShapeDtypeStruct((NP, H), jnp.float32),
          jax.ShapeDtypeStruct((NP, H), jnp.float32),
          jax.ShapeDtypeStruct((D, H), jnp.float32),
      ],
  )(x, W_msg, b_msg, W_u1, b_u1)


def _tc_post_kernel(acc_ref, deg_ref, x_ref, qa_ref, r_ref, a_ref, w2_ref,
                    b2_ref, out_ref):
  f32 = jnp.float32
  S = acc_ref[0] + acc_ref[1]
  deg = _dotg(deg_ref[...], jnp.ones((NW, 1), f32), 0, 0)
  h = jnp.dot(S, a_ref[...], preferred_element_type=f32) \
      + deg * qa_ref[...] + r_ref[...]
  h = jnp.maximum(h, 0.0)
  out_ref[...] = x_ref[...] \
      + jnp.dot(h, w2_ref[...], preferred_element_type=f32) + b2_ref[...]


def _tc_post(acc, deg, x, qa, r, a, W_u2, b_u2):
  grid = (NP // BLK,)
  full = lambda *shape: pl.BlockSpec(shape, lambda i: (0,) * len(shape))
  return pl.pallas_call(
      _tc_post_kernel,
      grid=grid,
      in_specs=[
          pl.BlockSpec((NC, BLK, D), lambda i: (0, i, 0)),
          pl.BlockSpec((NW, BLK), lambda i: (0, i)),
          pl.BlockSpec((BLK, D), lambda i: (i, 0)),
          pl.BlockSpec((BLK, H), lambda i: (i, 0)),
          pl.BlockSpec((BLK, H), lambda i: (i, 0)),
          full(D, H),
          full(H, D),
          full(1, D),
      ],
      out_specs=pl.BlockSpec((BLK, D), lambda i: (i, 0)),
      out_shape=jax.ShapeDtypeStruct((N, D), jnp.float32),
  )(acc, deg, x, qa, r, a, W_u2, b_u2)


def kernel(x, edge_index, W_msg, b_msg, W_u1, b_u1, W_u2, b_u2):
  zeros_acc = jnp.zeros((RPT, D), jnp.float32)
  acc, deg = _sc_segment_sum(x, edge_index, zeros_acc)
  qa, r, a = _tc_pre(x, W_msg, b_msg.reshape(1, -1), W_u1,
                     b_u1.reshape(1, -1))
  return _tc_post(acc, deg, x, qa, r, a, W_u2, b_u2.reshape(1, -1))
